# initial kernel scaffold (unmeasured)
import jax
import jax.numpy as jnp
from jax import lax
from jax.experimental import pallas as pl
from jax.experimental.pallas import tpu as pltpu


def kernel(x, k, Wp):
    B, H, W, C = x.shape
    N_GLOBAL = (2 * H) * (2 * W)

    def body(x_ref, k_ref, wp_ref, out_ref, pad_ref, stats_ref,
             send_sems, recv_sems):
        mx = lax.axis_index("x")
        my = lax.axis_index("y")
        x_nbr = (1 - mx, my)
        y_nbr = (mx, 1 - my)
        diag = (1 - mx, 1 - my)

        barrier = pltpu.get_barrier_semaphore()
        for nbr in (x_nbr, y_nbr, diag):
            pl.semaphore_signal(barrier, inc=1, device_id=nbr,
                                device_id_type=pl.DeviceIdType.MESH)
        pl.semaphore_wait(barrier, 3)

        src_row = (1 - mx) * (H - 1)
        dst_row = mx * (H + 1)
        row_send = pltpu.make_async_remote_copy(
            src_ref=x_ref.at[:, pl.ds(src_row, 1), :, :],
            dst_ref=pad_ref.at[:, pl.ds(dst_row, 1), pl.ds(1, W), :],
            send_sem=send_sems.at[0],
            recv_sem=recv_sems.at[0],
            device_id=x_nbr,
            device_id_type=pl.DeviceIdType.MESH,
        )
        row_send.start()

        src_col = (1 - my) * (W - 1)
        dst_col = my * (W + 1)
        col_send = pltpu.make_async_remote_copy(
            src_ref=x_ref.at[:, :, pl.ds(src_col, 1), :],
            dst_ref=pad_ref.at[:, pl.ds(1, H), pl.ds(dst_col, 1), :],
            send_sem=send_sems.at[1],
            recv_sem=recv_sems.at[1],
            device_id=y_nbr,
            device_id_type=pl.DeviceIdType.MESH,
        )
        col_send.start()

        corner_send = pltpu.make_async_remote_copy(
            src_ref=x_ref.at[:, pl.ds(src_row, 1), pl.ds(src_col, 1), :],
            dst_ref=pad_ref.at[:, pl.ds(dst_row, 1), pl.ds(dst_col, 1), :],
            send_sem=send_sems.at[2],
            recv_sem=recv_sems.at[2],
            device_id=diag,
            device_id_type=pl.DeviceIdType.MESH,
        )
        corner_send.start()

        xl = x_ref[...]
        stats_ref[0, 0] = jnp.sum(xl, axis=(1, 2))
        stats_ref[0, 1] = jnp.sum(xl * xl, axis=(1, 2))

        stats_sends = []
        for i, nbr in enumerate((x_nbr, y_nbr, diag)):
            s = pltpu.make_async_remote_copy(
                src_ref=stats_ref.at[0],
                dst_ref=stats_ref.at[1 + i],
                send_sem=send_sems.at[3 + i],
                recv_sem=recv_sems.at[3 + i],
                device_id=nbr,
                device_id_type=pl.DeviceIdType.MESH,
            )
            s.start()
            stats_sends.append(s)

        pad_ref[:, 1:H + 1, 1:W + 1, :] = xl
        er_src = mx * (H - 1)
        er_dst = mx * (H + 1)
        ec_src = my * (W - 1)
        ec_dst = my * (W + 1)
        pad_ref[:, pl.ds(er_dst, 1), 1:W + 1, :] = \
            x_ref[:, pl.ds(er_src, 1), :, :]
        pad_ref[:, 1:H + 1, pl.ds(ec_dst, 1), :] = \
            x_ref[:, :, pl.ds(ec_src, 1), :]
        pad_ref[:, pl.ds(er_dst, 1), pl.ds(ec_dst, 1), :] = \
            x_ref[:, pl.ds(er_src, 1), pl.ds(ec_src, 1), :]

        row_recv = pltpu.make_async_remote_copy(
            src_ref=x_ref.at[:, pl.ds(src_row, 1), :, :],
            dst_ref=pad_ref.at[:, pl.ds((1 - mx) * (H + 1), 1), pl.ds(1, W), :],
            send_sem=send_sems.at[0],
            recv_sem=recv_sems.at[0],
            device_id=x_nbr,
            device_id_type=pl.DeviceIdType.MESH,
        )
        row_recv.wait_recv()
        col_recv = pltpu.make_async_remote_copy(
            src_ref=x_ref.at[:, :, pl.ds(src_col, 1), :],
            dst_ref=pad_ref.at[:, pl.ds(1, H), pl.ds((1 - my) * (W + 1), 1), :],
            send_sem=send_sems.at[1],
            recv_sem=recv_sems.at[1],
            device_id=y_nbr,
            device_id_type=pl.DeviceIdType.MESH,
        )
        col_recv.wait_recv()
        corner_recv = pltpu.make_async_remote_copy(
            src_ref=x_ref.at[:, pl.ds(src_row, 1), pl.ds(src_col, 1), :],
            dst_ref=pad_ref.at[:, pl.ds((1 - mx) * (H + 1), 1),
                               pl.ds((1 - my) * (W + 1), 1), :],
            send_sem=send_sems.at[2],
            recv_sem=recv_sems.at[2],
            device_id=diag,
            device_id_type=pl.DeviceIdType.MESH,
        )
        corner_recv.wait_recv()

        pad_ref[:, pl.ds(er_dst, 1), pl.ds((1 - my) * (W + 1), 1), :] = \
            pad_ref[:, pl.ds(1 + mx * (H - 1), 1),
                    pl.ds((1 - my) * (W + 1), 1), :]
        pad_ref[:, pl.ds((1 - mx) * (H + 1), 1), pl.ds(ec_dst, 1), :] = \
            pad_ref[:, pl.ds((1 - mx) * (H + 1), 1),
                    pl.ds(1 + my * (W - 1), 1), :]

        kv = k_ref[...]
        conv = jnp.zeros((B, H, W, C), jnp.float32)
        for di in range(3):
            for dj in range(3):
                conv = conv + pad_ref[:, di:di + H, dj:dj + W, :] * kv[di, dj]

        for i in range(3):
            r = pltpu.make_async_remote_copy(
                src_ref=stats_ref.at[0],
                dst_ref=stats_ref.at[1 + i],
                send_sem=send_sems.at[3 + i],
                recv_sem=recv_sems.at[3 + i],
                device_id=diag,
                device_id_type=pl.DeviceIdType.MESH,
            )
            r.wait_recv()

        tot = (stats_ref[0] + stats_ref[1]) + (stats_ref[2] + stats_ref[3])
        mean = tot[0] / N_GLOBAL
        var = tot[1] / N_GLOBAL - mean * mean
        rstd = lax.rsqrt(var + 1e-5)
        ksum = jnp.sum(kv, axis=(0, 1))
        convh = (conv - (mean * ksum)[:, None, None, :]) \
            * rstd[:, None, None, :]

        a = convh * (1.0 / (1.0 + jnp.exp(-convh)))
        proj = jnp.dot(a.reshape(B * H * W, C), wp_ref[...],
                       preferred_element_type=jnp.float32)
        out_ref[...] = xl + proj.reshape(B, H, W, C)

        row_send.wait_send()
        col_send.wait_send()
        corner_send.wait_send()
        for s in stats_sends:
            s.wait_send()

    return pl.pallas_call(
        body,
        out_shape=jax.ShapeDtypeStruct((B, H, W, C), jnp.float32),
        in_specs=[
            pl.BlockSpec(memory_space=pltpu.VMEM),
            pl.BlockSpec(memory_space=pltpu.VMEM),
            pl.BlockSpec(memory_space=pltpu.VMEM),
        ],
        out_specs=pl.BlockSpec(memory_space=pltpu.VMEM),
        scratch_shapes=[
            pltpu.VMEM((B, H + 2, W + 2, C), jnp.float32),
            pltpu.VMEM((4, 2, B, C), jnp.float32),
            pltpu.SemaphoreType.DMA((6,)),
            pltpu.SemaphoreType.DMA((6,)),
        ],
        compiler_params=pltpu.CompilerParams(collective_id=0),
    )(x, k, Wp)


# baseline (device time: 61584 ns/iter reference)
import jax
import jax.numpy as jnp
from jax import lax
from jax.experimental import pallas as pl
from jax.experimental.pallas import tpu as pltpu


def kernel(x, k, Wp):
    B, H, W, C = x.shape
    WC = W * C
    N_GLOBAL = (2 * H) * (2 * W)

    def body(x_ref, k_ref, wp_ref, out_ref,
             srow_ref, rrow_ref, scol_ref, rcol_ref, scor_ref, rcor_ref,
             stats_ref, send_sems, recv_sems):
        mx = lax.axis_index("x")
        my = lax.axis_index("y")
        x_nbr = (1 - mx, my)
        y_nbr = (mx, 1 - my)
        diag = (1 - mx, 1 - my)
        on_x0 = mx == 0
        on_y0 = my == 0

        barrier = pltpu.get_barrier_semaphore()
        for nbr in (x_nbr, y_nbr, diag):
            pl.semaphore_signal(barrier, inc=1, device_id=nbr,
                                device_id_type=pl.DeviceIdType.MESH)
        pl.semaphore_wait(barrier, 3)

        xv = x_ref[...]
        srow_ref[...] = jnp.where(on_x0, xv[:, H - 1:H, :], xv[:, 0:1, :])
        scol = jnp.where(on_y0, xv[:, :, WC - C:], xv[:, :, 0:C])
        scol_ref[...] = scol
        scor_ref[...] = jnp.where(on_x0, scol[:, H - 1:H, :], scol[:, 0:1, :])

        row_rdma = pltpu.make_async_remote_copy(
            src_ref=srow_ref, dst_ref=rrow_ref,
            send_sem=send_sems.at[0], recv_sem=recv_sems.at[0],
            device_id=x_nbr, device_id_type=pl.DeviceIdType.MESH)
        row_rdma.start()
        col_rdma = pltpu.make_async_remote_copy(
            src_ref=scol_ref, dst_ref=rcol_ref,
            send_sem=send_sems.at[1], recv_sem=recv_sems.at[1],
            device_id=y_nbr, device_id_type=pl.DeviceIdType.MESH)
        col_rdma.start()
        cor_rdma = pltpu.make_async_remote_copy(
            src_ref=scor_ref, dst_ref=rcor_ref,
            send_sem=send_sems.at[2], recv_sem=recv_sems.at[2],
            device_id=diag, device_id_type=pl.DeviceIdType.MESH)
        cor_rdma.start()

        s1 = jnp.sum(xv, axis=1).reshape(B, W, C).sum(axis=1)
        s2 = jnp.sum(xv * xv, axis=1).reshape(B, W, C).sum(axis=1)
        stats_ref[0, 0] = s1
        stats_ref[0, 1] = s2
        stats_sends = []
        for i, nbr in enumerate((x_nbr, y_nbr, diag)):
            s = pltpu.make_async_remote_copy(
                src_ref=stats_ref.at[0], dst_ref=stats_ref.at[1 + i],
                send_sem=send_sems.at[3 + i], recv_sem=recv_sems.at[3 + i],
                device_id=nbr, device_id_type=pl.DeviceIdType.MESH)
            s.start()
            stats_sends.append(s)

        row_rdma.wait_recv()
        col_rdma.wait_recv()
        cor_rdma.wait_recv()

        for s in stats_sends:
            s.wait_recv()
        tot = (stats_ref[0] + stats_ref[1]) + (stats_ref[2] + stats_ref[3])
        mean = tot[0] / N_GLOBAL
        var = tot[1] / N_GLOBAL - mean * mean
        rstd = lax.rsqrt(var + 1e-5)

        kv = k_ref[...]
        ksum = jnp.sum(kv, axis=(0, 1))
        kvecs = [
            [jnp.broadcast_to(kv[di, dj].reshape(1, 1, C),
                              (1, W, C)).reshape(1, WC)
             for dj in range(3)]
            for di in range(3)
        ]
        wpv = wp_ref[...]

        for b in range(B):
            xb = x_ref[b]
            rrow = rrow_ref[b]
            rcol = rcol_ref[b]
            rcor = rcor_ref[b]

            lcol = jnp.where(on_y0, xb[:, 0:C], rcol)
            rcolp = jnp.where(on_y0, rcol, xb[:, WC - C:])
            trow = jnp.where(on_x0, xb[0:1, :], rrow)
            brow = jnp.where(on_x0, rrow, xb[H - 1:H, :])
            tl = jnp.where(on_x0, lcol[0:1, :],
                           jnp.where(on_y0, rrow[:, 0:C], rcor))
            tr = jnp.where(on_x0, rcolp[0:1, :],
                           jnp.where(on_y0, rcor, rrow[:, WC - C:]))
            bl = jnp.where(on_x0, jnp.where(on_y0, rrow[:, 0:C], rcor),
                           lcol[H - 1:H, :])
            br = jnp.where(on_x0, jnp.where(on_y0, rcor, rrow[:, WC - C:]),
                           rcolp[H - 1:H, :])

            vfull = jnp.concatenate([
                jnp.concatenate([tl, trow, tr], axis=1),
                jnp.concatenate([lcol, xb, rcolp], axis=1),
                jnp.concatenate([bl, brow, br], axis=1),
            ], axis=0)

            conv = jnp.zeros((H, WC), jnp.float32)
            for di in range(3):
                for dj in range(3):
                    conv = conv + (vfull[di:di + H, dj * C:dj * C + WC]
                                   * kvecs[di][dj])

            c3 = conv.reshape(H, W, C)
            ch3 = (c3 - (mean[b] * ksum)[None, None, :]) \
                * rstd[b][None, None, :]
            a3 = ch3 * (1.0 / (1.0 + jnp.exp(-ch3)))
            a2 = a3.reshape(H * W, C)
            proj = jnp.dot(a2, wpv, preferred_element_type=jnp.float32)
            out3 = proj.reshape(H, W, C) + xb.reshape(H, W, C)
            out_ref[b] = out3.reshape(H, WC)

        row_rdma.wait_send()
        col_rdma.wait_send()
        cor_rdma.wait_send()
        for s in stats_sends:
            s.wait_send()

    xf = x.reshape(B, H, WC)
    out = pl.pallas_call(
        body,
        out_shape=jax.ShapeDtypeStruct((B, H, WC), jnp.float32),
        in_specs=[
            pl.BlockSpec(memory_space=pltpu.VMEM),
            pl.BlockSpec(memory_space=pltpu.VMEM),
            pl.BlockSpec(memory_space=pltpu.VMEM),
        ],
        out_specs=pl.BlockSpec(memory_space=pltpu.VMEM),
        scratch_shapes=[
            pltpu.VMEM((B, 1, WC), jnp.float32),
            pltpu.VMEM((B, 1, WC), jnp.float32),
            pltpu.VMEM((B, H, C), jnp.float32),
            pltpu.VMEM((B, H, C), jnp.float32),
            pltpu.VMEM((B, 1, C), jnp.float32),
            pltpu.VMEM((B, 1, C), jnp.float32),
            pltpu.VMEM((4, 2, B, C), jnp.float32),
            pltpu.SemaphoreType.DMA((6,)),
            pltpu.SemaphoreType.DMA((6,)),
        ],
        compiler_params=pltpu.CompilerParams(collective_id=0),
    )(xf, k, Wp)
    return out.reshape(B, H, W, C)
